# baseline (speedup 1.0000x reference)
import jax
import jax.numpy as jnp
from jax import lax
from jax.experimental import pallas as pl
from jax.experimental.pallas import tpu as pltpu
from jax.experimental.pallas import tpu_sc as plsc

NV = 40962
NF = 81920
B = 4
CIN = 32
COUT = 32
C128 = B * CIN

NW = 32
FCH = 8
NCH_F = 320
FPW = NCH_F * FCH
VCH = 16
NCH_V = 84
NVP = NW * NCH_V * VCH
VPW = NCH_V * VCH

OFF_LV = NVP * 7
OFF_FC = 2 * NVP * 7
OFF_FV = 2 * NVP * 7 + NVP * 6
OFF_GV = NF * 9
OFF_EW = 2 * NF * 9
OFF_NS = 2 * NF * 9 + NF * 3


def _tr_body(x_ref, o_ref):
  o_ref[...] = x_ref[...].T


def _face_compute(t, gv0, gv1, gv2, ew_v, ns_v, rows_v, obuf):
  gvs = (gv0, gv1, gv2)

  def face(fi, c2):
    gw = []
    ewj = []
    nsj = []
    for j in range(3):
      for u in range(3):
        gw.append(plsc.bitcast(plsc.load_gather(
            gvs[j], [jnp.full((16,), t * 24 + fi * 3 + u, jnp.int32)]),
            jnp.float32))
      ewj.append(plsc.bitcast(plsc.load_gather(
          ew_v, [jnp.full((16,), t * 24 + fi * 3 + j, jnp.int32)]),
          jnp.float32))
      nsj.append(plsc.bitcast(plsc.load_gather(
          ns_v, [jnp.full((16,), t * 24 + fi * 3 + j, jnp.int32)]),
          jnp.float32))
    for cg in range(8):
      gj = []
      for j in range(3):
        r = rows_v[j * 24 + fi * 3, pl.ds(cg * 16, 16)]
        g = gw[j * 3] * r
        for u in range(1, 3):
          r = rows_v[j * 24 + fi * 3 + u, pl.ds(cg * 16, 16)]
          g = g + gw[j * 3 + u] * r
        gj.append(g)
      ae = ewj[0] * gj[0] + ewj[1] * gj[1] + ewj[2] * gj[2]
      an = nsj[0] * gj[0] + nsj[1] * gj[1] + nsj[2] * gj[2]
      obuf[fi, pl.ds(cg * 32, 32)] = plsc.pack(
          ae, an, format=plsc.PackFormat.INTERLEAVED)
    return c2

  lax.fori_loop(0, FCH, face, 0)


def _face_body(xt, gtab, fo,
               gc0, gc1, gc2, gv0, gv1, gv2, ew_v, ns_v,
               rows_a, rows_b, ob_a, ob_b,
               sem_a, sem_b, sem_oa, sem_ob):
  wid = lax.axis_index("s") * 2 + lax.axis_index("c")
  f_base = wid * FPW

  for j, (gc_v, gv_v) in enumerate(((gc0, gv0), (gc1, gv1), (gc2, gv2))):
    pltpu.sync_copy(gtab.at[pl.ds(j * NF * 3 + f_base * 3, FPW * 3)], gc_v)
    pltpu.sync_copy(
        gtab.at[pl.ds(OFF_GV + j * NF * 3 + f_base * 3, FPW * 3)], gv_v)
  pltpu.sync_copy(gtab.at[pl.ds(OFF_EW + f_base * 3, FPW * 3)], ew_v)
  pltpu.sync_copy(gtab.at[pl.ds(OFF_NS + f_base * 3, FPW * 3)], ns_v)

  gcs = (gc0, gc1, gc2)

  def issue(t, rows_v, sem):
    for j in range(3):
      pltpu.async_copy(xt.at[gcs[j].at[pl.ds(t * 24, 24)]],
                       rows_v.at[pl.ds(j * 24, 24)], sem)

  def wait(t, rows_v, sem):
    for j in range(3):
      pltpu.make_async_copy(
          xt.at[gcs[j].at[pl.ds(t * 24, 24)]],
          rows_v.at[pl.ds(j * 24, 24)], sem).wait()

  def owrite(t, ob, sem_o):
    pltpu.async_copy(ob, fo.at[pl.ds(f_base + t * FCH, FCH)], sem_o)

  def owait(ob, sem_o):
    pltpu.make_async_copy(ob, fo.at[pl.ds(f_base, FCH)], sem_o).wait()

  issue(0, rows_a, sem_a)

  def pair(s, carry):
    t = 2 * s
    t2 = t + 1
    issue(t2, rows_b, sem_b)
    wait(t, rows_a, sem_a)

    @pl.when(s > 0)
    def _():
      owait(ob_a, sem_oa)

    _face_compute(t, gv0, gv1, gv2, ew_v, ns_v, rows_a, ob_a)
    owrite(t, ob_a, sem_oa)

    @pl.when(s < NCH_F // 2 - 1)
    def _():
      issue(t + 2, rows_a, sem_a)

    wait(t2, rows_b, sem_b)

    @pl.when(s > 0)
    def _():
      owait(ob_b, sem_ob)

    _face_compute(t2, gv0, gv1, gv2, ew_v, ns_v, rows_b, ob_b)
    owrite(t2, ob_b, sem_ob)
    return carry

  lax.fori_loop(0, NCH_F // 2, pair, 0)
  owait(ob_a, sem_oa)
  owait(ob_b, sem_ob)


def _vert_compute(t, lv_v, fv_v, lrows_v, frows_v, obuf):
  def vert(vi, c2):
    wls = [plsc.bitcast(plsc.load_gather(
        lv_v, [jnp.full((16,), t * 112 + vi * 7 + k, jnp.int32)]),
        jnp.float32) for k in range(7)]
    wfs = [plsc.bitcast(plsc.load_gather(
        fv_v, [jnp.full((16,), t * 96 + vi * 6 + k, jnp.int32)]),
        jnp.float32) for k in range(6)]
    for cg in range(8):
      b = cg // 2
      off = (cg % 2) * 16
      orow = b * VCH + vi
      re, rn = plsc.unpack(frows_v[vi * 6, pl.ds(cg * 32, 32)],
                           format=plsc.PackFormat.INTERLEAVED)
      ae = wfs[0] * re
      an = wfs[0] * rn
      for k in range(1, 6):
        re, rn = plsc.unpack(frows_v[vi * 6 + k, pl.ds(cg * 32, 32)],
                             format=plsc.PackFormat.INTERLEAVED)
        ae = ae + wfs[k] * re
        an = an + wfs[k] * rn
      obuf[orow, pl.ds(off, 16)] = ae
      obuf[orow, pl.ds(32 + off, 16)] = an
      rl = lrows_v[vi * 7, pl.ds(cg * 16, 16)]
      al = wls[0] * rl
      for k in range(1, 7):
        rl = lrows_v[vi * 7 + k, pl.ds(cg * 16, 16)]
        al = al + wls[k] * rl
      obuf[orow, pl.ds(64 + off, 16)] = al
    return c2

  lax.fori_loop(0, VCH, vert, 0)


def _zero_tail(obuf):
  z = jnp.zeros((16,), jnp.float32)

  def row(r, c):
    obuf[r, pl.ds(96, 16)] = z
    obuf[r, pl.ds(112, 16)] = z
    return c

  lax.fori_loop(0, B * VCH, row, 0)


def _vert_body(xt, fo, lftab, res,
               lc_v, lv_v, fc_v, fv_v,
               lrows_a, lrows_b, frows_0, frows_1, frows_2, frows_3,
               obuf_a, obuf_b,
               sem_la, sem_lb, sem_f0, sem_f1, sem_f2, sem_f3,
               sem_oa, sem_ob):
  wid = lax.axis_index("s") * 2 + lax.axis_index("c")
  v_base = wid * VPW

  pltpu.sync_copy(lftab.at[pl.ds(v_base * 7, VPW * 7)], lc_v)
  pltpu.sync_copy(lftab.at[pl.ds(OFF_LV + v_base * 7, VPW * 7)], lv_v)
  pltpu.sync_copy(lftab.at[pl.ds(OFF_FC + v_base * 6, VPW * 6)], fc_v)
  pltpu.sync_copy(lftab.at[pl.ds(OFF_FV + v_base * 6, VPW * 6)], fv_v)
  _zero_tail(obuf_a)
  _zero_tail(obuf_b)

  lrows = (lrows_a, lrows_b)
  sem_l = (sem_la, sem_lb)
  frows = (frows_0, frows_1, frows_2, frows_3)
  sem_f = (sem_f0, sem_f1, sem_f2, sem_f3)
  obufs = (obuf_a, obuf_b)
  sem_o = (sem_oa, sem_ob)

  def lissue(t, q):
    pltpu.async_copy(xt.at[lc_v.at[pl.ds(t * 112, 112)]], lrows[q], sem_l[q])

  def lwait(t, q):
    pltpu.make_async_copy(
        xt.at[lc_v.at[pl.ds(t * 112, 112)]], lrows[q], sem_l[q]).wait()

  def fissue(t, q):
    pltpu.async_copy(fo.at[fc_v.at[pl.ds(t * 96, 96)]], frows[q], sem_f[q])

  def fwait(t, q):
    pltpu.make_async_copy(
        fo.at[fc_v.at[pl.ds(t * 96, 96)]], frows[q], sem_f[q]).wait()

  def owrite(t, q):
    for b in range(B):
      pltpu.async_copy(obufs[q].at[pl.ds(b * VCH, VCH)],
                       res.at[b, pl.ds(v_base + t * VCH, VCH)], sem_o[q])

  def owait(q):
    for b in range(B):
      pltpu.make_async_copy(obufs[q].at[pl.ds(b * VCH, VCH)],
                            res.at[b, pl.ds(v_base, VCH)], sem_o[q]).wait()

  lissue(0, 0)
  lissue(1, 1)
  for q in range(4):
    fissue(q, q)

  def quad(s, carry):
    for q in range(4):
      t = 4 * s + q
      fwait(t, q)
      lwait(t, q % 2)

      if q >= 2:
        owait(q % 2)
      else:
        @pl.when(s > 0)
        def _():
          owait(q % 2)

      _vert_compute(t, lv_v, fv_v, lrows[q % 2], frows[q], obufs[q % 2])
      owrite(t, q % 2)

      @pl.when(t + 2 < NCH_V)
      def _():
        lissue(t + 2, q % 2)

      @pl.when(t + 4 < NCH_V)
      def _():
        fissue(t + 4, q)
    return carry

  lax.fori_loop(0, NCH_V // 4, quad, 0)
  owait(0)
  owait(1)


def _mix_body(wi_ref, w3_ref, xt_ref, r_ref, o_ref):
  o_ref[0] = (
      jax.lax.dot_general(wi_ref[0], xt_ref[...],
                          dimension_numbers=(((1,), (1,)), ((), ())),
                          preferred_element_type=jnp.float32)
      + jax.lax.dot_general(w3_ref[...], r_ref[0],
                            dimension_numbers=(((1,), (1,)), ((), ())),
                            preferred_element_type=jnp.float32))


@jax.jit
def kernel(x, coeffs, G_rows, G_cols, G_vals, EW, NS,
           L_rows, L_cols, L_vals, F_rows, F_cols, F_vals):
  del G_rows, L_rows, F_rows

  x2 = x.reshape(C128, NV)
  xt = pl.pallas_call(
      _tr_body,
      grid=((NV + 511) // 512,),
      in_specs=[pl.BlockSpec((C128, 512), lambda i: (0, i))],
      out_specs=pl.BlockSpec((512, C128), lambda i: (i, 0)),
      out_shape=jax.ShapeDtypeStruct((NVP, C128), jnp.float32),
  )(x2)

  bc = lambda a: jax.lax.bitcast_convert_type(a, jnp.int32)
  gtab = jnp.concatenate([
      G_cols, bc(G_vals), bc(EW.reshape(NF * 3)), bc(NS.reshape(NF * 3))])
  z7 = jnp.zeros((NVP - NV) * 7, jnp.int32)
  z6 = jnp.zeros((NVP - NV) * 6, jnp.int32)
  lftab = jnp.concatenate([
      L_cols, z7, bc(L_vals), z7, F_cols, z6, bc(F_vals), z6])

  mesh = plsc.VectorSubcoreMesh(core_axis_name="c", subcore_axis_name="s",
                                num_cores=2, num_subcores=16)
  sc_params = pltpu.CompilerParams(needs_layout_passes=False,
                                   use_tc_tiling_on_sc=False)

  fo = pl.kernel(
      _face_body,
      out_type=jax.ShapeDtypeStruct((NF, 256), jnp.bfloat16),
      mesh=mesh,
      compiler_params=sc_params,
      scratch_types=[
          pltpu.VMEM((FPW * 3,), jnp.int32),
          pltpu.VMEM((FPW * 3,), jnp.int32),
          pltpu.VMEM((FPW * 3,), jnp.int32),
          pltpu.VMEM((FPW * 3,), jnp.int32),
          pltpu.VMEM((FPW * 3,), jnp.int32),
          pltpu.VMEM((FPW * 3,), jnp.int32),
          pltpu.VMEM((FPW * 3,), jnp.int32),
          pltpu.VMEM((FPW * 3,), jnp.int32),
          pltpu.VMEM((FCH * 9, C128), jnp.float32),
          pltpu.VMEM((FCH * 9, C128), jnp.float32),
          pltpu.VMEM((FCH, 256), jnp.bfloat16),
          pltpu.VMEM((FCH, 256), jnp.bfloat16),
          pltpu.SemaphoreType.DMA,
          pltpu.SemaphoreType.DMA,
          pltpu.SemaphoreType.DMA,
          pltpu.SemaphoreType.DMA,
      ],
  )(xt, gtab)

  res = pl.kernel(
      _vert_body,
      out_type=jax.ShapeDtypeStruct((B, NVP, C128), jnp.float32),
      mesh=mesh,
      compiler_params=sc_params,
      scratch_types=[
          pltpu.VMEM((VPW * 7,), jnp.int32),
          pltpu.VMEM((VPW * 7,), jnp.int32),
          pltpu.VMEM((VPW * 6,), jnp.int32),
          pltpu.VMEM((VPW * 6,), jnp.int32),
          pltpu.VMEM((VCH * 7, C128), jnp.float32),
          pltpu.VMEM((VCH * 7, C128), jnp.float32),
          pltpu.VMEM((VCH * 6, 256), jnp.bfloat16),
          pltpu.VMEM((VCH * 6, 256), jnp.bfloat16),
          pltpu.VMEM((VCH * 6, 256), jnp.bfloat16),
          pltpu.VMEM((VCH * 6, 256), jnp.bfloat16),
          pltpu.VMEM((B * VCH, C128), jnp.float32),
          pltpu.VMEM((B * VCH, C128), jnp.float32),
          pltpu.SemaphoreType.DMA,
          pltpu.SemaphoreType.DMA,
          pltpu.SemaphoreType.DMA,
          pltpu.SemaphoreType.DMA,
          pltpu.SemaphoreType.DMA,
          pltpu.SemaphoreType.DMA,
          pltpu.SemaphoreType.DMA,
          pltpu.SemaphoreType.DMA,
      ],
  )(xt, fo, lftab)

  w_idb = jnp.einsum('bc,oi->boci', jnp.eye(B, dtype=jnp.float32),
                     coeffs[:, :, 0]).reshape(B, COUT, C128)
  w_3 = jnp.pad(coeffs.transpose(0, 2, 1)[:, 1:, :].reshape(COUT, 3 * CIN),
                ((0, 0), (0, CIN)))

  nblk = (NV + 2047) // 2048
  out = pl.pallas_call(
      _mix_body,
      grid=(B, nblk),
      in_specs=[
          pl.BlockSpec((1, COUT, C128), lambda b, i: (b, 0, 0)),
          pl.BlockSpec((COUT, C128), lambda b, i: (0, 0)),
          pl.BlockSpec((2048, C128), lambda b, i: (i, 0)),
          pl.BlockSpec((1, 2048, C128), lambda b, i: (b, i, 0)),
      ],
      out_specs=pl.BlockSpec((1, COUT, 2048), lambda b, i: (b, 0, i)),
      out_shape=jax.ShapeDtypeStruct((B, COUT, NV), jnp.float32),
  )(w_idb, w_3, xt, res)

  return out


# speedup vs baseline: 1.2106x; 1.2106x over previous
import jax
import jax.numpy as jnp
from jax import lax
from jax.experimental import pallas as pl
from jax.experimental.pallas import tpu as pltpu
from jax.experimental.pallas import tpu_sc as plsc

NV = 40962
NF = 81920
B = 4
CIN = 32
COUT = 32
C128 = B * CIN

NW = 32
FCH = 8
NCH_F = 320
FPW = NCH_F * FCH
VCH = 16
NCH_V = 82
NVP = NW * NCH_V * VCH
VPW = NCH_V * VCH

OFF_LV = NVP * 7
OFF_FC = 2 * NVP * 7
OFF_FV = 2 * NVP * 7 + NVP * 6
OFF_GV = NF * 9
OFF_EW = 2 * NF * 9
OFF_NS = 2 * NF * 9 + NF * 3


def _tr_body(x_ref, o_ref):
  o_ref[...] = x_ref[...].T


def _face_compute(t, gv0, gv1, gv2, ew_v, ns_v, rows_v, obuf):
  gvs = (gv0, gv1, gv2)

  def face(fi, c2):
    gw = []
    ewj = []
    nsj = []
    for j in range(3):
      for u in range(3):
        gw.append(plsc.bitcast(plsc.load_gather(
            gvs[j], [jnp.full((16,), t * 24 + fi * 3 + u, jnp.int32)]),
            jnp.float32))
      ewj.append(plsc.bitcast(plsc.load_gather(
          ew_v, [jnp.full((16,), t * 24 + fi * 3 + j, jnp.int32)]),
          jnp.float32))
      nsj.append(plsc.bitcast(plsc.load_gather(
          ns_v, [jnp.full((16,), t * 24 + fi * 3 + j, jnp.int32)]),
          jnp.float32))
    for cg in range(8):
      gj = []
      for j in range(3):
        r = rows_v[j * 24 + fi * 3, pl.ds(cg * 16, 16)]
        g = gw[j * 3] * r
        for u in range(1, 3):
          r = rows_v[j * 24 + fi * 3 + u, pl.ds(cg * 16, 16)]
          g = g + gw[j * 3 + u] * r
        gj.append(g)
      ae = ewj[0] * gj[0] + ewj[1] * gj[1] + ewj[2] * gj[2]
      an = nsj[0] * gj[0] + nsj[1] * gj[1] + nsj[2] * gj[2]
      obuf[fi, pl.ds(cg * 32, 32)] = plsc.pack(
          ae, an, format=plsc.PackFormat.INTERLEAVED)
    return c2

  lax.fori_loop(0, FCH, face, 0)


def _face_body(xt, gtab, fo,
               gc0, gc1, gc2, gv0, gv1, gv2, ew_v, ns_v,
               rows_a, rows_b, ob_a, ob_b,
               sem_a, sem_b, sem_oa, sem_ob):
  wid = lax.axis_index("s") * 2 + lax.axis_index("c")
  f_base = wid * FPW

  for j, (gc_v, gv_v) in enumerate(((gc0, gv0), (gc1, gv1), (gc2, gv2))):
    pltpu.sync_copy(gtab.at[pl.ds(j * NF * 3 + f_base * 3, FPW * 3)], gc_v)
    pltpu.sync_copy(
        gtab.at[pl.ds(OFF_GV + j * NF * 3 + f_base * 3, FPW * 3)], gv_v)
  pltpu.sync_copy(gtab.at[pl.ds(OFF_EW + f_base * 3, FPW * 3)], ew_v)
  pltpu.sync_copy(gtab.at[pl.ds(OFF_NS + f_base * 3, FPW * 3)], ns_v)

  gcs = (gc0, gc1, gc2)

  def issue(t, rows_v, sem):
    for j in range(3):
      pltpu.async_copy(xt.at[gcs[j].at[pl.ds(t * 24, 24)]],
                       rows_v.at[pl.ds(j * 24, 24)], sem)

  def wait(t, rows_v, sem):
    for j in range(3):
      pltpu.make_async_copy(
          xt.at[gcs[j].at[pl.ds(t * 24, 24)]],
          rows_v.at[pl.ds(j * 24, 24)], sem).wait()

  def owrite(t, ob, sem_o):
    pltpu.async_copy(ob, fo.at[pl.ds(f_base + t * FCH, FCH)], sem_o)

  def owait(ob, sem_o):
    pltpu.make_async_copy(ob, fo.at[pl.ds(f_base, FCH)], sem_o).wait()

  issue(0, rows_a, sem_a)

  def pair(s, carry):
    t = 2 * s
    t2 = t + 1
    issue(t2, rows_b, sem_b)
    wait(t, rows_a, sem_a)

    @pl.when(s > 0)
    def _():
      owait(ob_a, sem_oa)

    _face_compute(t, gv0, gv1, gv2, ew_v, ns_v, rows_a, ob_a)
    owrite(t, ob_a, sem_oa)

    @pl.when(s < NCH_F // 2 - 1)
    def _():
      issue(t + 2, rows_a, sem_a)

    wait(t2, rows_b, sem_b)

    @pl.when(s > 0)
    def _():
      owait(ob_b, sem_ob)

    _face_compute(t2, gv0, gv1, gv2, ew_v, ns_v, rows_b, ob_b)
    owrite(t2, ob_b, sem_ob)
    return carry

  lax.fori_loop(0, NCH_F // 2, pair, 0)
  owait(ob_a, sem_oa)
  owait(ob_b, sem_ob)


def _vert_compute(t, lv_v, fv_v, lrows_v, frows_v, obuf):
  def vert(vi, c2):
    wls = [plsc.bitcast(plsc.load_gather(
        lv_v, [jnp.full((16,), t * 112 + vi * 7 + k, jnp.int32)]),
        jnp.float32) for k in range(7)]
    wfs = [plsc.bitcast(plsc.load_gather(
        fv_v, [jnp.full((16,), t * 96 + vi * 6 + k, jnp.int32)]),
        jnp.float32) for k in range(6)]
    for cg in range(8):
      b = cg // 2
      off = (cg % 2) * 16
      orow = b * VCH + vi
      re, rn = plsc.unpack(frows_v[vi * 6, pl.ds(cg * 32, 32)],
                           format=plsc.PackFormat.INTERLEAVED)
      ae = wfs[0] * re
      an = wfs[0] * rn
      for k in range(1, 6):
        re, rn = plsc.unpack(frows_v[vi * 6 + k, pl.ds(cg * 32, 32)],
                             format=plsc.PackFormat.INTERLEAVED)
        ae = ae + wfs[k] * re
        an = an + wfs[k] * rn
      obuf[orow, pl.ds(off, 16)] = ae
      obuf[orow, pl.ds(32 + off, 16)] = an
      rl = lrows_v[vi * 7, pl.ds(cg * 16, 16)]
      al = wls[0] * rl
      for k in range(1, 7):
        rl = lrows_v[vi * 7 + k, pl.ds(cg * 16, 16)]
        al = al + wls[k] * rl
      obuf[orow, pl.ds(64 + off, 16)] = al
    return c2

  lax.fori_loop(0, VCH, vert, 0)


def _zero_tail(obuf):
  z = jnp.zeros((16,), jnp.float32)

  def row(r, c):
    obuf[r, pl.ds(96, 16)] = z
    obuf[r, pl.ds(112, 16)] = z
    return c

  lax.fori_loop(0, B * VCH, row, 0)


def _vert_body(xt, fo, lftab, res,
               lc_v, lv_v, fc_v, fv_v,
               lrows_a, lrows_b, frows_a, frows_b,
               obuf_a, obuf_b,
               sem_la, sem_lb, sem_fa, sem_fb, sem_oa, sem_ob):
  wid = lax.axis_index("s") * 2 + lax.axis_index("c")
  v_base = wid * VPW

  pltpu.sync_copy(lftab.at[pl.ds(v_base * 7, VPW * 7)], lc_v)
  pltpu.sync_copy(lftab.at[pl.ds(OFF_LV + v_base * 7, VPW * 7)], lv_v)
  pltpu.sync_copy(lftab.at[pl.ds(OFF_FC + v_base * 6, VPW * 6)], fc_v)
  pltpu.sync_copy(lftab.at[pl.ds(OFF_FV + v_base * 6, VPW * 6)], fv_v)
  _zero_tail(obuf_a)
  _zero_tail(obuf_b)

  def issue(t, lrows_v, frows_v, sem_l, sem_f):
    pltpu.async_copy(xt.at[lc_v.at[pl.ds(t * 112, 112)]], lrows_v, sem_l)
    pltpu.async_copy(fo.at[fc_v.at[pl.ds(t * 96, 96)]], frows_v, sem_f)

  def wait(t, lrows_v, frows_v, sem_l, sem_f):
    pltpu.make_async_copy(
        xt.at[lc_v.at[pl.ds(t * 112, 112)]], lrows_v, sem_l).wait()
    pltpu.make_async_copy(
        fo.at[fc_v.at[pl.ds(t * 96, 96)]], frows_v, sem_f).wait()

  def owrite(t, obuf, sem_o):
    for b in range(B):
      pltpu.async_copy(obuf.at[pl.ds(b * VCH, VCH)],
                       res.at[b, pl.ds(v_base + t * VCH, VCH)], sem_o)

  def owait(obuf, sem_o):
    for b in range(B):
      pltpu.make_async_copy(obuf.at[pl.ds(b * VCH, VCH)],
                            res.at[b, pl.ds(v_base, VCH)], sem_o).wait()

  issue(0, lrows_a, frows_a, sem_la, sem_fa)

  def pair(s, carry):
    t = 2 * s
    t2 = t + 1
    issue(t2, lrows_b, frows_b, sem_lb, sem_fb)
    wait(t, lrows_a, frows_a, sem_la, sem_fa)

    @pl.when(s > 0)
    def _():
      owait(obuf_a, sem_oa)

    _vert_compute(t, lv_v, fv_v, lrows_a, frows_a, obuf_a)
    owrite(t, obuf_a, sem_oa)

    @pl.when(s < NCH_V // 2 - 1)
    def _():
      issue(t + 2, lrows_a, frows_a, sem_la, sem_fa)

    wait(t2, lrows_b, frows_b, sem_lb, sem_fb)

    @pl.when(s > 0)
    def _():
      owait(obuf_b, sem_ob)

    _vert_compute(t2, lv_v, fv_v, lrows_b, frows_b, obuf_b)
    owrite(t2, obuf_b, sem_ob)
    return carry

  lax.fori_loop(0, NCH_V // 2, pair, 0)
  owait(obuf_a, sem_oa)
  owait(obuf_b, sem_ob)


def _mix_body(wi_ref, w3_ref, xt_ref, r_ref, o_ref):
  o_ref[0] = (
      jax.lax.dot_general(wi_ref[0], xt_ref[...],
                          dimension_numbers=(((1,), (1,)), ((), ())),
                          preferred_element_type=jnp.float32)
      + jax.lax.dot_general(w3_ref[...], r_ref[0],
                            dimension_numbers=(((1,), (1,)), ((), ())),
                            preferred_element_type=jnp.float32))


@jax.jit
def kernel(x, coeffs, G_rows, G_cols, G_vals, EW, NS,
           L_rows, L_cols, L_vals, F_rows, F_cols, F_vals):
  del G_rows, L_rows, F_rows

  x2 = x.reshape(C128, NV)
  xt = pl.pallas_call(
      _tr_body,
      grid=((NV + 511) // 512,),
      in_specs=[pl.BlockSpec((C128, 512), lambda i: (0, i))],
      out_specs=pl.BlockSpec((512, C128), lambda i: (i, 0)),
      out_shape=jax.ShapeDtypeStruct((NVP, C128), jnp.float32),
  )(x2)

  bc = lambda a: jax.lax.bitcast_convert_type(a, jnp.int32)
  gtab = jnp.concatenate([
      G_cols, bc(G_vals), bc(EW.reshape(NF * 3)), bc(NS.reshape(NF * 3))])
  z7 = jnp.zeros((NVP - NV) * 7, jnp.int32)
  z6 = jnp.zeros((NVP - NV) * 6, jnp.int32)
  lftab = jnp.concatenate([
      L_cols, z7, bc(L_vals), z7, F_cols, z6, bc(F_vals), z6])

  mesh = plsc.VectorSubcoreMesh(core_axis_name="c", subcore_axis_name="s",
                                num_cores=2, num_subcores=16)
  sc_params = pltpu.CompilerParams(needs_layout_passes=False,
                                   use_tc_tiling_on_sc=False)

  fo = pl.kernel(
      _face_body,
      out_type=jax.ShapeDtypeStruct((NF, 256), jnp.bfloat16),
      mesh=mesh,
      compiler_params=sc_params,
      scratch_types=[
          pltpu.VMEM((FPW * 3,), jnp.int32),
          pltpu.VMEM((FPW * 3,), jnp.int32),
          pltpu.VMEM((FPW * 3,), jnp.int32),
          pltpu.VMEM((FPW * 3,), jnp.int32),
          pltpu.VMEM((FPW * 3,), jnp.int32),
          pltpu.VMEM((FPW * 3,), jnp.int32),
          pltpu.VMEM((FPW * 3,), jnp.int32),
          pltpu.VMEM((FPW * 3,), jnp.int32),
          pltpu.VMEM((FCH * 9, C128), jnp.float32),
          pltpu.VMEM((FCH * 9, C128), jnp.float32),
          pltpu.VMEM((FCH, 256), jnp.bfloat16),
          pltpu.VMEM((FCH, 256), jnp.bfloat16),
          pltpu.SemaphoreType.DMA,
          pltpu.SemaphoreType.DMA,
          pltpu.SemaphoreType.DMA,
          pltpu.SemaphoreType.DMA,
      ],
  )(xt, gtab)

  res = pl.kernel(
      _vert_body,
      out_type=jax.ShapeDtypeStruct((B, NVP, C128), jnp.float32),
      mesh=mesh,
      compiler_params=sc_params,
      scratch_types=[
          pltpu.VMEM((VPW * 7,), jnp.int32),
          pltpu.VMEM((VPW * 7,), jnp.int32),
          pltpu.VMEM((VPW * 6,), jnp.int32),
          pltpu.VMEM((VPW * 6,), jnp.int32),
          pltpu.VMEM((VCH * 7, C128), jnp.float32),
          pltpu.VMEM((VCH * 7, C128), jnp.float32),
          pltpu.VMEM((VCH * 6, 256), jnp.bfloat16),
          pltpu.VMEM((VCH * 6, 256), jnp.bfloat16),
          pltpu.VMEM((B * VCH, C128), jnp.float32),
          pltpu.VMEM((B * VCH, C128), jnp.float32),
          pltpu.SemaphoreType.DMA,
          pltpu.SemaphoreType.DMA,
          pltpu.SemaphoreType.DMA,
          pltpu.SemaphoreType.DMA,
          pltpu.SemaphoreType.DMA,
          pltpu.SemaphoreType.DMA,
      ],
  )(xt, fo, lftab)

  w_idb = jnp.einsum('bc,oi->boci', jnp.eye(B, dtype=jnp.float32),
                     coeffs[:, :, 0]).reshape(B, COUT, C128)
  w_3 = jnp.pad(coeffs.transpose(0, 2, 1)[:, 1:, :].reshape(COUT, 3 * CIN),
                ((0, 0), (0, CIN)))

  nblk = (NV + 2047) // 2048
  out = pl.pallas_call(
      _mix_body,
      grid=(B, nblk),
      in_specs=[
          pl.BlockSpec((1, COUT, C128), lambda b, i: (b, 0, 0)),
          pl.BlockSpec((COUT, C128), lambda b, i: (0, 0)),
          pl.BlockSpec((2048, C128), lambda b, i: (i, 0)),
          pl.BlockSpec((1, 2048, C128), lambda b, i: (b, i, 0)),
      ],
      out_specs=pl.BlockSpec((1, COUT, 2048), lambda b, i: (b, 0, i)),
      out_shape=jax.ShapeDtypeStruct((B, COUT, NV), jnp.float32),
  )(w_idb, w_3, xt, res)

  return out
